# baseline (speedup 1.0000x reference)
import functools

import jax
import jax.numpy as jnp
from jax import lax
from jax.experimental import pallas as pl
from jax.experimental.pallas import tpu as pltpu
from jax.experimental.pallas import tpu_sc as plsc

EMB = 64
B = 4096 * 50
NC, NS = 2, 16
NW = NC * NS
BPW = B // NW
CHUNK = 128
NCHUNK = BPW // CHUNK
NBUF = 5
NG = NCHUNK // NBUF

_mesh = plsc.VectorSubcoreMesh(core_axis_name="c", subcore_axis_name="s")


@functools.partial(
    pl.kernel,
    mesh=_mesh,
    out_type=jax.ShapeDtypeStruct((B, EMB), jnp.float32),
    compiler_params=pltpu.CompilerParams(use_tc_tiling_on_sc=False),
    scratch_types=[
        pltpu.VMEM((NCHUNK, CHUNK), jnp.int32),
        *[pltpu.VMEM((CHUNK, EMB), jnp.float32) for _ in range(NBUF)],
        *[pltpu.SemaphoreType.DMA for _ in range(2 * NBUF)],
    ],
)
def _gather_kernel(idx_hbm, table_hbm, out_hbm, idx_v, *bufs_and_sems):
    rows = bufs_and_sems[:NBUF]
    gsem = bufs_and_sems[NBUF:2 * NBUF]
    wsem = bufs_and_sems[2 * NBUF:]

    wid = lax.axis_index("s") * NC + lax.axis_index("c")
    base = wid * BPW

    pltpu.sync_copy(idx_hbm.at[wid], idx_v)

    def fire_gather(ci, b):
        pltpu.make_async_copy(
            table_hbm.at[idx_v.at[ci]], rows[b], gsem[b]).start()

    def wait_gather(ci, b):
        pltpu.make_async_copy(
            table_hbm.at[idx_v.at[ci]], rows[b], gsem[b]).wait()

    def fire_write(ci, b):
        pltpu.make_async_copy(
            rows[b], out_hbm.at[pl.ds(base + ci * CHUNK, CHUNK)],
            wsem[b]).start()

    def wait_write(ci, b):
        pltpu.make_async_copy(
            rows[b], out_hbm.at[pl.ds(base + ci * CHUNK, CHUNK)],
            wsem[b]).wait()

    for b in range(NBUF):
        fire_gather(b, b)

    def group_body(g, carry):
        for b in range(NBUF):
            ci = g * NBUF + b
            wait_gather(ci, b)
            fire_write(ci, b)
            wait_write(ci, b)
            fire_gather(ci + NBUF, b)
        return carry

    lax.fori_loop(0, NG - 1, group_body, 0)

    for b in range(NBUF):
        ci = (NG - 1) * NBUF + b
        wait_gather(ci, b)
        fire_write(ci, b)
    for b in range(NBUF):
        ci = (NG - 1) * NBUF + b
        wait_write(ci, b)


def kernel(vocab_ids, node_embs):
    idx = vocab_ids.reshape(NW, NCHUNK, CHUNK).astype(jnp.int32)
    out = _gather_kernel(idx, node_embs)
    return out.reshape(vocab_ids.shape + (EMB,))
